# Initial kernel scaffold; baseline (speedup 1.0000x reference)
import jax
import jax.numpy as jnp
from jax.experimental import pallas as pl


def kernel(year, month, day, year_table, month_table, day_table):
    raise NotImplementedError("write your pallas kernel here")


# speedup vs baseline: 8.7957x; 8.7957x over previous
import functools

import jax
import jax.numpy as jnp
from jax import lax
from jax.experimental import pallas as pl
from jax.experimental.pallas import tpu as pltpu
from jax.experimental.pallas import tpu_sc as plsc

NUM_CORES = 2
NUM_SUBCORES = 16
NUM_TILES = NUM_CORES * NUM_SUBCORES
LANES = 16
CHUNK = 128


def _comb_body(m_ref, d_ref, o_ref):
    m = m_ref[...]
    d = d_ref[...]
    o_ref[...] = (m[:, None, :] + d[None, :, :]).reshape(13 * 32, 128)


def _make_comb(month_table, day_table):
    return pl.pallas_call(
        _comb_body,
        out_shape=jax.ShapeDtypeStruct((13 * 32, 128), jnp.float32),
    )(month_table, day_table)


def _make_sc_kernel(n, hidden):
    per_tile = n // NUM_TILES
    n_chunks = per_tile // CHUNK
    mesh = plsc.VectorSubcoreMesh(
        core_axis_name="c", subcore_axis_name="s",
        num_cores=NUM_CORES, num_subcores=NUM_SUBCORES,
    )

    @functools.partial(
        pl.kernel,
        out_type=jax.ShapeDtypeStruct((n, hidden), jnp.float32),
        mesh=mesh,
        scratch_types=[
            pltpu.VMEM((CHUNK,), jnp.int32),
            pltpu.VMEM((CHUNK,), jnp.int32),
            pltpu.VMEM((CHUNK,), jnp.int32),
            pltpu.VMEM((CHUNK,), jnp.int32),
            pltpu.VMEM((CHUNK, hidden), jnp.float32),
            pltpu.VMEM((CHUNK, hidden), jnp.float32),
            pltpu.SemaphoreType.DMA,
            pltpu.SemaphoreType.DMA,
        ],
    )
    def _sc(year_hbm, month_hbm, day_hbm, ytab_hbm, ctab_hbm, out_hbm,
            yidx_v, midx_v, didx_v, md_v, rows_y, rows_c, sem_y, sem_c):
        wid = lax.axis_index("s") * NUM_CORES + lax.axis_index("c")
        base0 = wid * per_tile

        @pl.loop(0, n_chunks)
        def _chunk_loop(c):
            base = base0 + c * CHUNK
            pltpu.sync_copy(year_hbm.at[pl.ds(base, CHUNK)], yidx_v)
            pltpu.sync_copy(month_hbm.at[pl.ds(base, CHUNK)], midx_v)
            pltpu.sync_copy(day_hbm.at[pl.ds(base, CHUNK)], didx_v)
            for k in range(CHUNK // LANES):
                s = pl.ds(k * LANES, LANES)
                md_v[s] = midx_v[s] * 32 + didx_v[s]
            cpy_y = pltpu.async_copy(ytab_hbm.at[yidx_v], rows_y, sem_y)
            cpy_c = pltpu.async_copy(ctab_hbm.at[md_v], rows_c, sem_c)
            cpy_y.wait()
            cpy_c.wait()

            @pl.loop(0, CHUNK)
            def _row_loop(i):
                for k in range(hidden // LANES):
                    s = pl.ds(k * LANES, LANES)
                    rows_y[i, s] = rows_y[i, s] + rows_c[i, s]

            pltpu.sync_copy(rows_y, out_hbm.at[pl.ds(base, CHUNK)])

    return _sc


def kernel(year, month, day, year_table, month_table, day_table):
    b, l = year.shape
    hidden = year_table.shape[1]
    n = b * l
    yidx = year.reshape(n).astype(jnp.int32)
    midx = month.reshape(n).astype(jnp.int32)
    didx = day.reshape(n).astype(jnp.int32)
    comb = _make_comb(month_table.astype(jnp.float32),
                      day_table.astype(jnp.float32))
    sc = _make_sc_kernel(n, hidden)
    out = sc(yidx, midx, didx, year_table, comb)
    return out.reshape(b, l, hidden)


# speedup vs baseline: 8.8150x; 1.0022x over previous
import functools

import jax
import jax.numpy as jnp
from jax import lax
from jax.experimental import pallas as pl
from jax.experimental.pallas import tpu as pltpu
from jax.experimental.pallas import tpu_sc as plsc

NUM_CORES = 2
NUM_SUBCORES = 16
NUM_TILES = NUM_CORES * NUM_SUBCORES
LANES = 16
CHUNK = 128


def _comb_body(m_ref, d_ref, o_ref):
    m = m_ref[...]
    d = d_ref[...]
    o_ref[...] = (m[:, None, :] + d[None, :, :]).reshape(13 * 32, 128)


def _make_comb(month_table, day_table):
    return pl.pallas_call(
        _comb_body,
        out_shape=jax.ShapeDtypeStruct((13 * 32, 128), jnp.float32),
    )(month_table, day_table)


def _make_sc_kernel(n, hidden):
    per_tile = n // NUM_TILES
    n_chunks = per_tile // CHUNK
    mesh = plsc.VectorSubcoreMesh(
        core_axis_name="c", subcore_axis_name="s",
        num_cores=NUM_CORES, num_subcores=NUM_SUBCORES,
    )

    @functools.partial(
        pl.kernel,
        out_type=jax.ShapeDtypeStruct((n, hidden), jnp.float32),
        mesh=mesh,
        scratch_types=[
            pltpu.VMEM((CHUNK,), jnp.int32),
            pltpu.VMEM((CHUNK,), jnp.int32),
            pltpu.VMEM((CHUNK,), jnp.int32),
            pltpu.VMEM((CHUNK,), jnp.int32),
            pltpu.VMEM((CHUNK, hidden), jnp.float32),
            pltpu.VMEM((CHUNK, hidden), jnp.float32),
            pltpu.SemaphoreType.DMA,
            pltpu.SemaphoreType.DMA,
        ],
    )
    def _sc(year_hbm, month_hbm, day_hbm, ytab_hbm, ctab_hbm, out_hbm,
            yidx_v, midx_v, didx_v, md_v, rows_y, rows_c, sem_y, sem_c):
        wid = lax.axis_index("s") * NUM_CORES + lax.axis_index("c")
        base0 = wid * per_tile

        @pl.loop(0, n_chunks)
        def _chunk_loop(c):
            base = base0 + c * CHUNK
            pltpu.sync_copy(year_hbm.at[pl.ds(base, CHUNK)], yidx_v)
            pltpu.sync_copy(month_hbm.at[pl.ds(base, CHUNK)], midx_v)
            pltpu.sync_copy(day_hbm.at[pl.ds(base, CHUNK)], didx_v)
            for k in range(CHUNK // LANES):
                s = pl.ds(k * LANES, LANES)
                md_v[s] = midx_v[s] * 32 + didx_v[s]
            cpy_y = pltpu.async_copy(ytab_hbm.at[yidx_v], rows_y, sem_y)
            cpy_c = pltpu.async_copy(ctab_hbm.at[md_v], rows_c, sem_c)
            cpy_y.wait()
            cpy_c.wait()

            @pl.loop(0, CHUNK)
            def _row_loop(i):
                for k in range(hidden // LANES):
                    s = pl.ds(k * LANES, LANES)
                    plsc.addupdate(rows_y.at[i, s], rows_c[i, s])

            pltpu.sync_copy(rows_y, out_hbm.at[pl.ds(base, CHUNK)])

    return _sc


def kernel(year, month, day, year_table, month_table, day_table):
    b, l = year.shape
    hidden = year_table.shape[1]
    n = b * l
    yidx = year.reshape(n).astype(jnp.int32)
    midx = month.reshape(n).astype(jnp.int32)
    didx = day.reshape(n).astype(jnp.int32)
    comb = _make_comb(month_table.astype(jnp.float32),
                      day_table.astype(jnp.float32))
    sc = _make_sc_kernel(n, hidden)
    out = sc(yidx, midx, didx, year_table, comb)
    return out.reshape(b, l, hidden)


# speedup vs baseline: 13.9050x; 1.5774x over previous
import functools

import jax
import jax.numpy as jnp
from jax import lax
from jax.experimental import pallas as pl
from jax.experimental.pallas import tpu as pltpu
from jax.experimental.pallas import tpu_sc as plsc

NUM_CORES = 2
NUM_SUBCORES = 16
NUM_TILES = NUM_CORES * NUM_SUBCORES
LANES = 16
CHUNK = 128


def _comb_body(m_ref, d_ref, o_ref):
    m = m_ref[...]
    d = d_ref[...]
    o_ref[...] = (m[:, None, :] + d[None, :, :]).reshape(13 * 32, 128)


def _make_comb(month_table, day_table):
    return pl.pallas_call(
        _comb_body,
        out_shape=jax.ShapeDtypeStruct((13 * 32, 128), jnp.float32),
    )(month_table, day_table)


def _make_sc_kernel(n, hidden):
    per_tile = n // NUM_TILES
    n_chunks = per_tile // CHUNK
    assert n_chunks % 2 == 0 and n_chunks >= 6
    mesh = plsc.VectorSubcoreMesh(
        core_axis_name="c", subcore_axis_name="s",
        num_cores=NUM_CORES, num_subcores=NUM_SUBCORES,
    )

    idx_t = pltpu.VMEM((CHUNK,), jnp.int32)
    rows_t = pltpu.VMEM((CHUNK, hidden), jnp.float32)

    @functools.partial(
        pl.kernel,
        out_type=jax.ShapeDtypeStruct((n, hidden), jnp.float32),
        mesh=mesh,
        scratch_types=[idx_t] * 8 + [rows_t] * 4 + [pltpu.SemaphoreType.DMA] * 6,
    )
    def _sc(year_hbm, month_hbm, day_hbm, ytab_hbm, ctab_hbm, out_hbm,
            yi0, yi1, mi0, mi1, di0, di1, md0, md1,
            ry0, ry1, rc0, rc1,
            semi0, semi1, semg0, semg1, semo0, semo1):
        yidx, midx, didx, md = (yi0, yi1), (mi0, mi1), (di0, di1), (md0, md1)
        rows_y, rows_c = (ry0, ry1), (rc0, rc1)
        sem_idx, sem_g, sem_out = (semi0, semi1), (semg0, semg1), (semo0, semo1)

        wid = lax.axis_index("s") * NUM_CORES + lax.axis_index("c")
        base0 = wid * per_tile

        def issue_idx(c, b):
            base = base0 + c * CHUNK
            pltpu.async_copy(year_hbm.at[pl.ds(base, CHUNK)], yidx[b], sem_idx[b])
            pltpu.async_copy(month_hbm.at[pl.ds(base, CHUNK)], midx[b], sem_idx[b])
            pltpu.async_copy(day_hbm.at[pl.ds(base, CHUNK)], didx[b], sem_idx[b])

        def wait_idx(b):
            pltpu.make_async_copy(year_hbm.at[pl.ds(0, CHUNK)], yidx[b], sem_idx[b]).wait()
            pltpu.make_async_copy(month_hbm.at[pl.ds(0, CHUNK)], midx[b], sem_idx[b]).wait()
            pltpu.make_async_copy(day_hbm.at[pl.ds(0, CHUNK)], didx[b], sem_idx[b]).wait()

        def compute_md(b):
            for k in range(CHUNK // LANES):
                s = pl.ds(k * LANES, LANES)
                md[b][s] = midx[b][s] * 32 + didx[b][s]

        def issue_gather(b):
            pltpu.async_copy(ytab_hbm.at[yidx[b]], rows_y[b], sem_g[b])
            pltpu.async_copy(ctab_hbm.at[md[b]], rows_c[b], sem_g[b])

        def wait_gather(b):
            pltpu.make_async_copy(ytab_hbm.at[yidx[b]], rows_y[b], sem_g[b]).wait()
            pltpu.make_async_copy(ctab_hbm.at[md[b]], rows_c[b], sem_g[b]).wait()

        def issue_out(c, b):
            base = base0 + c * CHUNK
            pltpu.async_copy(rows_y[b], out_hbm.at[pl.ds(base, CHUNK)], sem_out[b])

        def wait_out(b):
            pltpu.make_async_copy(rows_y[b], out_hbm.at[pl.ds(0, CHUNK)], sem_out[b]).wait()

        def add_rows(b):
            @pl.loop(0, CHUNK)
            def _row_loop(i):
                for k in range(hidden // LANES):
                    s = pl.ds(k * LANES, LANES)
                    plsc.addupdate(rows_y[b].at[i, s], rows_c[b][i, s])

        def step(c, b, wait_prev_out, next_gather, next_idx):
            nb = 1 - b
            if next_gather:
                wait_idx(nb)
                compute_md(nb)
                if wait_prev_out:
                    wait_out(nb)
                issue_gather(nb)
            wait_gather(b)
            if next_idx:
                issue_idx(c + 2, b)
            add_rows(b)
            issue_out(c, b)

        issue_idx(0, 0)
        issue_idx(1, 1)
        wait_idx(0)
        compute_md(0)
        issue_gather(0)
        step(0, 0, False, True, True)

        @pl.loop(1, n_chunks - 3, step=2)
        def _main(c):
            step(c, 1, True, True, True)
            step(c + 1, 0, True, True, True)

        step(n_chunks - 3, 1, True, True, True)
        step(n_chunks - 2, 0, True, True, False)
        step(n_chunks - 1, 1, True, False, False)
        wait_out(0)
        wait_out(1)

    return _sc


def kernel(year, month, day, year_table, month_table, day_table):
    b, l = year.shape
    hidden = year_table.shape[1]
    n = b * l
    yidx = year.reshape(n).astype(jnp.int32)
    midx = month.reshape(n).astype(jnp.int32)
    didx = day.reshape(n).astype(jnp.int32)
    comb = _make_comb(month_table.astype(jnp.float32),
                      day_table.astype(jnp.float32))
    sc = _make_sc_kernel(n, hidden)
    out = sc(yidx, midx, didx, year_table, comb)
    return out.reshape(b, l, hidden)
